# baseline (speedup 1.0000x reference)
import jax
import jax.numpy as jnp
from jax.experimental import pallas as pl
from jax.experimental.pallas import tpu as pltpu

_BUFF = 16
_PPF = 80
_PATCH2 = 9
_C = 128
_H = 128
_W = 128
_DS = 4
_FLS_H = 512.0
_FLS_W = 512.0
_R_MIN = 0.5
_R_MAX = 30.0
_FOV_H = 130.0
_PK = _C * _PATCH2
_PW = _H // _DS

_HPR = 2
_CH = _C // _HPR
_NPA = (_BUFF - 1) * _HPR
_NBUF = 6
_LAG = 3


def _stream_body(scal_ref, fmap_hbm, imap_hbm, f1b2, ib2, f1o2, io2,
                 pooled_ref, stage, fvm, sem_in, sem_out, sem_f):
    li = scal_ref[0]

    cp_f = pltpu.make_async_copy(fmap_hbm, fvm, sem_f.at[0])
    cp_f.start()

    def _chunk_refs(k):
        if k >= 2 * _NPA:
            h = k - 2 * _NPA
            return imap_hbm.at[h], io2.at[li * _HPR + h], k % _NBUF
        arr_in, arr_out = (f1b2, f1o2) if k < _NPA else (ib2, io2)
        j = k if k < _NPA else k - _NPA
        v, half = j // _HPR, j % _HPR
        row = v + (v >= li).astype(jnp.int32)
        idx = row * _HPR + half
        return arr_in.at[idx], arr_out.at[idx], k % _NBUF

    def _start_in(k):
        src, _, slot = _chunk_refs(k)
        pltpu.make_async_copy(src, stage.at[slot], sem_in.at[slot]).start()

    def _wait_in(k):
        src, _, slot = _chunk_refs(k)
        pltpu.make_async_copy(src, stage.at[slot], sem_in.at[slot]).wait()

    def _start_out(k):
        _, dst, slot = _chunk_refs(k)
        pltpu.make_async_copy(stage.at[slot], dst, sem_out.at[slot]).start()

    def _wait_out(k):
        _, dst, slot = _chunk_refs(k)
        pltpu.make_async_copy(stage.at[slot], dst, sem_out.at[slot]).wait()

    _NTOT = 2 * _NPA + _HPR
    for k in range(_NTOT + _LAG):
        if k < _NTOT:
            if k >= _NBUF:
                _wait_out(k - _NBUF)
            _start_in(k)
        if k >= _LAG:
            j = k - _LAG
            _wait_in(j)
            _start_out(j)
        if k == _LAG:
            cp_f.wait()
            pltpu.make_async_copy(
                fvm, f1o2.at[pl.ds(li * _HPR, _HPR)], sem_f.at[1]).start()

            def _pool(ci, carry):
                h = ci // (_CH // 8)
                s0 = (ci % (_CH // 8)) * 8
                xs = fvm[h, pl.ds(s0, 8)]
                a = xs.reshape(8, _PW, _DS, _W).sum(axis=2)
                b = a.reshape(8, _PW, _PW, _DS).sum(axis=3)
                pooled_ref[pl.ds(h * _CH + s0, 8)] = b * (1.0 / (_DS * _DS))
                return carry

            jax.lax.fori_loop(0, _C // 8, _pool, 0)

    for k in range(_NTOT - _NBUF, _NTOT):
        _wait_out(k)

    pltpu.make_async_copy(fvm, f1o2.at[pl.ds(li * _HPR, _HPR)],
                          sem_f.at[1]).wait()


def _small_body(scal_ref, ts_ref, coords_ref, pooled_ref, patches_ref,
                f2b_ref, pb_ref, psb_ref, tb_ref, sfb_ref,
                f2o_ref, po_ref, pso_ref, to_ref, sfo_ref):
    r = pl.program_id(0)
    li = scal_ref[0]
    fn = scal_ref[1]

    @pl.when(r == li)
    def _():
        f2o_ref[0] = pooled_ref[0]
        po_ref[0] = patches_ref[0]
        xy = coords_ref[0]
        rp = xy[1:2, :] * ((_R_MAX - _R_MIN) / _FLS_H) + _R_MIN
        th = (xy[0:1, :] * (1.0 / _FLS_W) - 0.5) * (_FOV_H * jnp.pi / 180.0)
        pso_ref[0] = jnp.concatenate(
            [rp, th, jnp.zeros((1, _PPF), jnp.float32)], axis=0)
        sfo_ref[0] = jnp.full((1, _PPF), fn, dtype=jnp.int32)

    @pl.when(r != li)
    def _():
        f2o_ref[0] = f2b_ref[0]
        po_ref[0] = pb_ref[0]
        pso_ref[0] = psb_ref[0]
        sfo_ref[0] = sfb_ref[0]

    @pl.when(r == 0)
    def _():
        lanes = jax.lax.broadcasted_iota(jnp.int32, (1, _BUFF), 1)
        to_ref[...] = jnp.where(lanes == li, ts_ref[0, 0], tb_ref[...])


def kernel(fmap, imap, patches, coords, time_stamp, frame_n,
           fmap1_buf, fmap2_buf, imap_buf, patches_buf,
           patch_state_buf, time_buf, source_frame_buf):
    frame_n = jnp.asarray(frame_n, jnp.int32)
    li = frame_n % _BUFF
    scal = jnp.stack([li, frame_n])

    f32 = jnp.float32
    hbm = pl.BlockSpec(memory_space=pltpu.MemorySpace.HBM)
    smem = pl.BlockSpec(memory_space=pltpu.SMEM)
    vmem = pl.BlockSpec(memory_space=pltpu.VMEM)

    f1b2 = fmap1_buf.reshape(_BUFF * _HPR, _CH, _H, _W)
    ib2 = imap_buf.reshape(_BUFF * _HPR, _CH, _H, _W)

    f1o2, io2, pooled = pl.pallas_call(
        _stream_body,
        in_specs=[smem, hbm, hbm, hbm, hbm],
        out_specs=[hbm, hbm, vmem],
        out_shape=[
            jax.ShapeDtypeStruct((_BUFF * _HPR, _CH, _H, _W), f32),
            jax.ShapeDtypeStruct((_BUFF * _HPR, _CH, _H, _W), f32),
            jax.ShapeDtypeStruct((_C, _PW, _PW), f32),
        ],
        scratch_shapes=[
            pltpu.VMEM((_NBUF, _CH, _H, _W), f32),
            pltpu.VMEM((_HPR, _CH, _H, _W), f32),
            pltpu.SemaphoreType.DMA((_NBUF,)),
            pltpu.SemaphoreType.DMA((_NBUF,)),
            pltpu.SemaphoreType.DMA((2,)),
        ],
    )(scal, fmap.reshape(_HPR, _CH, _H, _W), imap.reshape(_HPR, _CH, _H, _W),
      f1b2, ib2)

    fmap1_new = f1o2.reshape(_BUFF, _C, _H, _W)
    imap_new = io2.reshape(_BUFF, _C, _H, _W)

    pooled2 = pooled.reshape(1, _C, _PW * _PW)
    f2b2 = fmap2_buf.reshape(_BUFF, _C, _PW * _PW)
    pflat = patches.reshape(1, _PPF, _PK)
    pbflat = patches_buf.reshape(_BUFF, _PPF, _PK)
    coords2 = coords[0].T.reshape(1, 2, _PPF)
    ts2 = time_stamp.reshape(1, 1)
    ps3 = jnp.swapaxes(patch_state_buf, 1, 2)
    tb2 = time_buf.reshape(1, _BUFF)
    sf3 = source_frame_buf.reshape(_BUFF, 1, _PPF)

    small = pl.pallas_call(
        _small_body,
        grid_spec=pltpu.PrefetchScalarGridSpec(
            num_scalar_prefetch=1,
            grid=(_BUFF,),
            in_specs=[
                pl.BlockSpec((1, 1), lambda r, s: (0, 0)),
                pl.BlockSpec((1, 2, _PPF), lambda r, s: (0, 0, 0)),
                pl.BlockSpec((1, _C, _PW * _PW), lambda r, s: (0, 0, 0)),
                pl.BlockSpec((1, _PPF, _PK), lambda r, s: (0, 0, 0)),
                pl.BlockSpec((1, _C, _PW * _PW), lambda r, s: (r, 0, 0)),
                pl.BlockSpec((1, _PPF, _PK), lambda r, s: (r, 0, 0)),
                pl.BlockSpec((1, 3, _PPF), lambda r, s: (r, 0, 0)),
                pl.BlockSpec((1, _BUFF), lambda r, s: (0, 0)),
                pl.BlockSpec((1, 1, _PPF), lambda r, s: (r, 0, 0)),
            ],
            out_specs=[
                pl.BlockSpec((1, _C, _PW * _PW), lambda r, s: (r, 0, 0)),
                pl.BlockSpec((1, _PPF, _PK), lambda r, s: (r, 0, 0)),
                pl.BlockSpec((1, 3, _PPF), lambda r, s: (r, 0, 0)),
                pl.BlockSpec((1, _BUFF), lambda r, s: (0, 0)),
                pl.BlockSpec((1, 1, _PPF), lambda r, s: (r, 0, 0)),
            ],
        ),
        out_shape=[
            jax.ShapeDtypeStruct((_BUFF, _C, _PW * _PW), f32),
            jax.ShapeDtypeStruct((_BUFF, _PPF, _PK), f32),
            jax.ShapeDtypeStruct((_BUFF, 3, _PPF), f32),
            jax.ShapeDtypeStruct((1, _BUFF), f32),
            jax.ShapeDtypeStruct((_BUFF, 1, _PPF), jnp.int32),
        ],
    )
    f2new, pnew, psnew, tnew, sfnew = small(scal, ts2, coords2, pooled2,
                                            pflat, f2b2, pbflat, ps3,
                                            tb2, sf3)

    return (fmap1_new,
            f2new.reshape(_BUFF, _C, _PW, _PW),
            imap_new,
            pnew.reshape(_BUFF, _PPF, _C, _PATCH2),
            jnp.swapaxes(psnew, 1, 2),
            tnew.reshape(_BUFF),
            sfnew.reshape(_BUFF, _PPF))


# speedup vs baseline: 1.0011x; 1.0011x over previous
import jax
import jax.numpy as jnp
from jax.experimental import pallas as pl
from jax.experimental.pallas import tpu as pltpu

_BUFF = 16
_PPF = 80
_PATCH2 = 9
_C = 128
_H = 128
_W = 128
_DS = 4
_FLS_H = 512.0
_FLS_W = 512.0
_R_MIN = 0.5
_R_MAX = 30.0
_FOV_H = 130.0
_PK = _C * _PATCH2
_PW = _H // _DS

_HPR = 2
_CH = _C // _HPR
_NPA = (_BUFF - 1) * _HPR
_NBUF = 6
_LAG = 3


def _stream_body(scal_ref, fmap_hbm, imap_hbm, f1b2, ib2, f1o2, io2,
                 pooled_ref, stage, fvm, sem_in, sem_out, sem_f):
    li = scal_ref[0]

    cp_f = pltpu.make_async_copy(fmap_hbm, fvm, sem_f.at[0])
    cp_f.start()

    def _chunk_refs(k):
        if k >= 2 * _NPA:
            h = k - 2 * _NPA
            return imap_hbm.at[h], io2.at[li * _HPR + h], k % _NBUF
        arr_in, arr_out = (f1b2, f1o2) if k < _NPA else (ib2, io2)
        j = k if k < _NPA else k - _NPA
        v, half = j // _HPR, j % _HPR
        row = v + (v >= li).astype(jnp.int32)
        idx = row * _HPR + half
        return arr_in.at[idx], arr_out.at[idx], k % _NBUF

    def _start_in(k):
        src, _, slot = _chunk_refs(k)
        pltpu.async_copy(src, stage.at[slot], sem_in.at[slot], priority=0)

    def _wait_in(k):
        src, _, slot = _chunk_refs(k)
        pltpu.make_async_copy(src, stage.at[slot], sem_in.at[slot]).wait()

    def _start_out(k):
        _, dst, slot = _chunk_refs(k)
        pltpu.async_copy(stage.at[slot], dst, sem_out.at[slot], priority=1)

    def _wait_out(k):
        _, dst, slot = _chunk_refs(k)
        pltpu.make_async_copy(stage.at[slot], dst, sem_out.at[slot]).wait()

    _NTOT = 2 * _NPA + _HPR
    for k in range(_NTOT + _LAG):
        if k < _NTOT:
            if k >= _NBUF:
                _wait_out(k - _NBUF)
            _start_in(k)
        if k >= _LAG:
            j = k - _LAG
            _wait_in(j)
            _start_out(j)
        if k == _LAG:
            cp_f.wait()
            pltpu.make_async_copy(
                fvm, f1o2.at[pl.ds(li * _HPR, _HPR)], sem_f.at[1]).start()

            def _pool(ci, carry):
                h = ci // (_CH // 8)
                s0 = (ci % (_CH // 8)) * 8
                xs = fvm[h, pl.ds(s0, 8)]
                a = xs.reshape(8, _PW, _DS, _W).sum(axis=2)
                b = a.reshape(8, _PW, _PW, _DS).sum(axis=3)
                pooled_ref[pl.ds(h * _CH + s0, 8)] = b * (1.0 / (_DS * _DS))
                return carry

            jax.lax.fori_loop(0, _C // 8, _pool, 0)

    for k in range(_NTOT - _NBUF, _NTOT):
        _wait_out(k)

    pltpu.make_async_copy(fvm, f1o2.at[pl.ds(li * _HPR, _HPR)],
                          sem_f.at[1]).wait()


def _small_body(scal_ref, ts_ref, coords_ref, pooled_ref, patches_ref,
                f2b_ref, pb_ref, psb_ref, tb_ref, sfb_ref,
                f2o_ref, po_ref, pso_ref, to_ref, sfo_ref):
    r = pl.program_id(0)
    li = scal_ref[0]
    fn = scal_ref[1]

    @pl.when(r == li)
    def _():
        f2o_ref[0] = pooled_ref[0]
        po_ref[0] = patches_ref[0]
        xy = coords_ref[0]
        rp = xy[1:2, :] * ((_R_MAX - _R_MIN) / _FLS_H) + _R_MIN
        th = (xy[0:1, :] * (1.0 / _FLS_W) - 0.5) * (_FOV_H * jnp.pi / 180.0)
        pso_ref[0] = jnp.concatenate(
            [rp, th, jnp.zeros((1, _PPF), jnp.float32)], axis=0)
        sfo_ref[0] = jnp.full((1, _PPF), fn, dtype=jnp.int32)

    @pl.when(r != li)
    def _():
        f2o_ref[0] = f2b_ref[0]
        po_ref[0] = pb_ref[0]
        pso_ref[0] = psb_ref[0]
        sfo_ref[0] = sfb_ref[0]

    @pl.when(r == 0)
    def _():
        lanes = jax.lax.broadcasted_iota(jnp.int32, (1, _BUFF), 1)
        to_ref[...] = jnp.where(lanes == li, ts_ref[0, 0], tb_ref[...])


def kernel(fmap, imap, patches, coords, time_stamp, frame_n,
           fmap1_buf, fmap2_buf, imap_buf, patches_buf,
           patch_state_buf, time_buf, source_frame_buf):
    frame_n = jnp.asarray(frame_n, jnp.int32)
    li = frame_n % _BUFF
    scal = jnp.stack([li, frame_n])

    f32 = jnp.float32
    hbm = pl.BlockSpec(memory_space=pltpu.MemorySpace.HBM)
    smem = pl.BlockSpec(memory_space=pltpu.SMEM)
    vmem = pl.BlockSpec(memory_space=pltpu.VMEM)

    f1b2 = fmap1_buf.reshape(_BUFF * _HPR, _CH, _H, _W)
    ib2 = imap_buf.reshape(_BUFF * _HPR, _CH, _H, _W)

    f1o2, io2, pooled = pl.pallas_call(
        _stream_body,
        in_specs=[smem, hbm, hbm, hbm, hbm],
        out_specs=[hbm, hbm, vmem],
        out_shape=[
            jax.ShapeDtypeStruct((_BUFF * _HPR, _CH, _H, _W), f32),
            jax.ShapeDtypeStruct((_BUFF * _HPR, _CH, _H, _W), f32),
            jax.ShapeDtypeStruct((_C, _PW, _PW), f32),
        ],
        scratch_shapes=[
            pltpu.VMEM((_NBUF, _CH, _H, _W), f32),
            pltpu.VMEM((_HPR, _CH, _H, _W), f32),
            pltpu.SemaphoreType.DMA((_NBUF,)),
            pltpu.SemaphoreType.DMA((_NBUF,)),
            pltpu.SemaphoreType.DMA((2,)),
        ],
    )(scal, fmap.reshape(_HPR, _CH, _H, _W), imap.reshape(_HPR, _CH, _H, _W),
      f1b2, ib2)

    fmap1_new = f1o2.reshape(_BUFF, _C, _H, _W)
    imap_new = io2.reshape(_BUFF, _C, _H, _W)

    pooled2 = pooled.reshape(1, _C, _PW * _PW)
    f2b2 = fmap2_buf.reshape(_BUFF, _C, _PW * _PW)
    pflat = patches.reshape(1, _PPF, _PK)
    pbflat = patches_buf.reshape(_BUFF, _PPF, _PK)
    coords2 = coords[0].T.reshape(1, 2, _PPF)
    ts2 = time_stamp.reshape(1, 1)
    ps3 = jnp.swapaxes(patch_state_buf, 1, 2)
    tb2 = time_buf.reshape(1, _BUFF)
    sf3 = source_frame_buf.reshape(_BUFF, 1, _PPF)

    small = pl.pallas_call(
        _small_body,
        grid_spec=pltpu.PrefetchScalarGridSpec(
            num_scalar_prefetch=1,
            grid=(_BUFF,),
            in_specs=[
                pl.BlockSpec((1, 1), lambda r, s: (0, 0)),
                pl.BlockSpec((1, 2, _PPF), lambda r, s: (0, 0, 0)),
                pl.BlockSpec((1, _C, _PW * _PW), lambda r, s: (0, 0, 0)),
                pl.BlockSpec((1, _PPF, _PK), lambda r, s: (0, 0, 0)),
                pl.BlockSpec((1, _C, _PW * _PW), lambda r, s: (r, 0, 0)),
                pl.BlockSpec((1, _PPF, _PK), lambda r, s: (r, 0, 0)),
                pl.BlockSpec((1, 3, _PPF), lambda r, s: (r, 0, 0)),
                pl.BlockSpec((1, _BUFF), lambda r, s: (0, 0)),
                pl.BlockSpec((1, 1, _PPF), lambda r, s: (r, 0, 0)),
            ],
            out_specs=[
                pl.BlockSpec((1, _C, _PW * _PW), lambda r, s: (r, 0, 0)),
                pl.BlockSpec((1, _PPF, _PK), lambda r, s: (r, 0, 0)),
                pl.BlockSpec((1, 3, _PPF), lambda r, s: (r, 0, 0)),
                pl.BlockSpec((1, _BUFF), lambda r, s: (0, 0)),
                pl.BlockSpec((1, 1, _PPF), lambda r, s: (r, 0, 0)),
            ],
        ),
        out_shape=[
            jax.ShapeDtypeStruct((_BUFF, _C, _PW * _PW), f32),
            jax.ShapeDtypeStruct((_BUFF, _PPF, _PK), f32),
            jax.ShapeDtypeStruct((_BUFF, 3, _PPF), f32),
            jax.ShapeDtypeStruct((1, _BUFF), f32),
            jax.ShapeDtypeStruct((_BUFF, 1, _PPF), jnp.int32),
        ],
    )
    f2new, pnew, psnew, tnew, sfnew = small(scal, ts2, coords2, pooled2,
                                            pflat, f2b2, pbflat, ps3,
                                            tb2, sf3)

    return (fmap1_new,
            f2new.reshape(_BUFF, _C, _PW, _PW),
            imap_new,
            pnew.reshape(_BUFF, _PPF, _C, _PATCH2),
            jnp.swapaxes(psnew, 1, 2),
            tnew.reshape(_BUFF),
            sfnew.reshape(_BUFF, _PPF))
